# Initial kernel scaffold; baseline (speedup 1.0000x reference)
import functools

import jax
import jax.numpy as jnp
from jax import lax
from jax.experimental import pallas as pl
from jax.experimental.pallas import tpu as pltpu
from jax.experimental.pallas import tpu_sc as plsc

NN = 10000
EE = 320000
DD = 128
HH = 32
GG = 64
NP = 10240
CH = 128
NBLK = EE // CH
NWORK = 32
PERT = NP // 16

_mesh = plsc.VectorSubcoreMesh(core_axis_name="c", subcore_axis_name="s")


def _wid(c, s):
    return s * 2 + c


def _nblk(wid):
    base = NBLK // NWORK
    rem = NBLK % NWORK
    return base + (wid < rem).astype(jnp.int32)


@functools.partial(
    pl.kernel,
    out_type=jax.ShapeDtypeStruct((2, NP), jnp.float32),
    mesh=_mesh,
    scratch_types=[
        pltpu.VMEM((CH,), jnp.int32),
        pltpu.VMEM((CH,), jnp.float32),
        pltpu.VMEM((PERT,), jnp.float32),
        pltpu.VMEM_SHARED((NP,), jnp.float32),
    ],
)
def _k1_deg(dst_hbm, z640_hbm, deg_out, idx_d, ones_v, zb, deg_s):
    c = lax.axis_index("c")
    s = lax.axis_index("s")
    wid = _wid(c, s)
    for i in range(CH // 16):
        ones_v[pl.ds(16 * i, 16)] = jnp.full((16,), 1.0, jnp.float32)
    pltpu.sync_copy(z640_hbm, zb)
    pltpu.sync_copy(zb, deg_s.at[pl.ds(s * PERT, PERT)])
    plsc.subcore_barrier()

    def body(j, carry):
        off = (wid + NWORK * j) * CH
        pltpu.sync_copy(dst_hbm.at[pl.ds(off, CH)], idx_d)
        pltpu.sync_copy(ones_v, deg_s.at[idx_d], add=True)
        return carry

    lax.fori_loop(0, _nblk(wid), body, 0)
    plsc.subcore_barrier()
    pltpu.sync_copy(deg_s.at[pl.ds(s * PERT, PERT)],
                    deg_out.at[c].at[pl.ds(s * PERT, PERT)])


def _k2_body(deg_ref, x_ref, w1_ref, hd_ref, dis_ref):
    deg = deg_ref[0] + deg_ref[1] + 1.0
    dis = lax.rsqrt(deg)
    dis_ref[...] = dis
    h = jnp.dot(x_ref[...], w1_ref[...], preferred_element_type=jnp.float32)
    hd_ref[...] = h * dis.reshape(-1, 1)


def _k2(deg_p, x_p, w1):
    rb = 1024
    grid = NP // rb
    return pl.pallas_call(
        _k2_body,
        grid=(grid,),
        in_specs=[
            pl.BlockSpec((2, rb // 128, 128), lambda i: (0, i, 0)),
            pl.BlockSpec((rb, DD), lambda i: (i, 0)),
            pl.BlockSpec((DD, HH), lambda i: (0, 0)),
        ],
        out_specs=[
            pl.BlockSpec((rb, HH), lambda i: (i, 0)),
            pl.BlockSpec((rb // 128, 128), lambda i: (i, 0)),
        ],
        out_shape=[
            jax.ShapeDtypeStruct((NP, HH), jnp.float32),
            jax.ShapeDtypeStruct((NP // 128, 128), jnp.float32),
        ],
    )(deg_p.reshape(2, NP // 128, 128), x_p, w1)


@functools.partial(
    pl.kernel,
    out_type=jax.ShapeDtypeStruct((2, NP, HH), jnp.float32),
    mesh=_mesh,
    scratch_types=[
        pltpu.VMEM((CH,), jnp.int32),
        pltpu.VMEM((CH,), jnp.int32),
        pltpu.VMEM((CH, HH), jnp.float32),
        pltpu.VMEM((CH, HH), jnp.float32),
        pltpu.SemaphoreType.DMA,
        pltpu.VMEM_SHARED((NP, HH), jnp.float32),
    ],
)
def _k3_msg(hd_hbm, src_hbm, dst_hbm, zrow_hbm, acc_out,
            idx_s, idx_d, rows, zb, sem, acc_s):
    c = lax.axis_index("c")
    s = lax.axis_index("s")
    wid = _wid(c, s)
    pltpu.sync_copy(zrow_hbm, zb)
    for j in range(PERT // CH):
        pltpu.sync_copy(zb, acc_s.at[pl.ds(s * PERT + j * CH, CH)])
    plsc.subcore_barrier()

    def body(j, carry):
        off = (wid + NWORK * j) * CH
        pltpu.sync_copy(src_hbm.at[pl.ds(off, CH)], idx_s)
        pltpu.sync_copy(dst_hbm.at[pl.ds(off, CH)], idx_d)
        pltpu.async_copy(hd_hbm.at[idx_s], rows, sem).wait()
        pltpu.sync_copy(rows, acc_s.at[idx_d], add=True)
        return carry

    lax.fori_loop(0, _nblk(wid), body, 0)
    plsc.subcore_barrier()
    pltpu.sync_copy(acc_s.at[pl.ds(s * PERT, PERT)],
                    acc_out.at[c].at[pl.ds(s * PERT, PERT)])


def _k4_body(acc_ref, hd_ref, dis_ref, b1_ref, w2_ref, wlr_ref, td_ref):
    dis = dis_ref[...]
    discol = dis.reshape(-1, 1)
    pre = (acc_ref[0] + acc_ref[1] + hd_ref[...]) * discol + b1_ref[...]
    a1 = jnp.maximum(pre, 0.0)
    z = jnp.dot(a1, w2_ref[...], preferred_element_type=jnp.float32)
    t = jnp.sum(z * wlr_ref[...], axis=1, keepdims=True)
    td_ref[...] = (t * discol).reshape(dis.shape)


def _k4(acc_p, hd, dis2d, b1r, w2, wlr):
    rb = 1024
    grid = NP // rb
    return pl.pallas_call(
        _k4_body,
        grid=(grid,),
        in_specs=[
            pl.BlockSpec((2, rb, HH), lambda i: (0, i, 0)),
            pl.BlockSpec((rb, HH), lambda i: (i, 0)),
            pl.BlockSpec((rb // 128, 128), lambda i: (i, 0)),
            pl.BlockSpec((1, HH), lambda i: (0, 0)),
            pl.BlockSpec((HH, HH), lambda i: (0, 0)),
            pl.BlockSpec((1, HH), lambda i: (0, 0)),
        ],
        out_specs=pl.BlockSpec((rb // 128, 128), lambda i: (i, 0)),
        out_shape=jax.ShapeDtypeStruct((NP // 128, 128), jnp.float32),
    )(acc_p, hd, dis2d, b1r, w2, wlr)


@functools.partial(
    pl.kernel,
    out_type=jax.ShapeDtypeStruct((2, 2 * GG), jnp.float32),
    mesh=_mesh,
    scratch_types=[
        pltpu.VMEM((CH,), jnp.int32),
        pltpu.VMEM((CH,), jnp.int32),
        pltpu.VMEM((CH,), jnp.float32),
        pltpu.VMEM((CH,), jnp.float32),
        pltpu.VMEM((CH,), jnp.float32),
        pltpu.VMEM((CH,), jnp.float32),
        pltpu.VMEM((CH,), jnp.int32),
        pltpu.VMEM((NP,), jnp.float32),
        pltpu.VMEM((PERT,), jnp.float32),
        pltpu.VMEM_SHARED((NP,), jnp.float32),
        pltpu.VMEM_SHARED((2 * GG,), jnp.float32),
    ],
)
def _k5_pool(td_hbm, dis_hbm, batch_hbm, src_hbm, dst_hbm, z640_hbm, gsum_out,
             idx_s, idx_d, vals, qbuf, rbuf, dbuf, bbuf, td_v, zb, r_s, bins_s):
    c = lax.axis_index("c")
    s = lax.axis_index("s")
    wid = _wid(c, s)
    pltpu.sync_copy(td_hbm, td_v)
    pltpu.sync_copy(z640_hbm, zb)
    pltpu.sync_copy(zb, r_s.at[pl.ds(s * PERT, PERT)])

    @pl.when(s == 0)
    def _():
        pltpu.sync_copy(zb.at[pl.ds(0, 2 * GG)], bins_s)

    plsc.subcore_barrier()

    def body(j, carry):
        off = (wid + NWORK * j) * CH
        pltpu.sync_copy(src_hbm.at[pl.ds(off, CH)], idx_s)
        pltpu.sync_copy(dst_hbm.at[pl.ds(off, CH)], idx_d)
        for f in range(CH // 16):
            sl = pl.ds(16 * f, 16)
            vals[sl] = plsc.load_gather(td_v, [idx_s[sl]])
        pltpu.sync_copy(vals, r_s.at[idx_d], add=True)
        return carry

    lax.fori_loop(0, _nblk(wid), body, 0)
    plsc.subcore_barrier()

    coef = jnp.where(c == 0, 1.0, 0.0).astype(jnp.float32)
    for k in range(PERT // CH):
        off = s * PERT + k * CH
        pltpu.sync_copy(r_s.at[pl.ds(off, CH)], rbuf)
        pltpu.sync_copy(dis_hbm.at[pl.ds(off, CH)], dbuf)
        pltpu.sync_copy(batch_hbm.at[pl.ds(off, CH)], bbuf)
        for f in range(CH // 16):
            sl = pl.ds(16 * f, 16)
            qbuf[sl] = dbuf[sl] * (rbuf[sl] + coef * td_v[pl.ds(off + 16 * f, 16)])
        pltpu.sync_copy(qbuf, bins_s.at[bbuf], add=True)
    plsc.subcore_barrier()

    @pl.when(s == 0)
    def _():
        pltpu.sync_copy(bins_s, gsum_out.at[c])


def _k6_body(gsum_ref, batch_ref, b2_ref, wlr_ref, bl_ref, out_ref):
    bm = batch_ref[...].reshape(NP, 1)
    gio = lax.broadcasted_iota(jnp.int32, (NP, GG), 1)
    onehot = jnp.where(bm == gio, 1.0, 0.0)
    cnt = jnp.sum(onehot, axis=0, keepdims=True)
    gs = gsum_ref[...]
    tot = gs[0:1, :GG] + gs[1:2, :GG]
    sbw = jnp.sum(b2_ref[...] * wlr_ref[...])
    out = tot / jnp.maximum(cnt, 1.0) + jnp.where(cnt > 0.0, sbw, 0.0)
    out_ref[...] = out + bl_ref[...]


def _k6(gsum_p, batch2d, b2r, wlr, blr):
    return pl.pallas_call(
        _k6_body,
        in_specs=[
            pl.BlockSpec((2, 2 * GG), lambda: (0, 0)),
            pl.BlockSpec((NP // 128, 128), lambda: (0, 0)),
            pl.BlockSpec((1, HH), lambda: (0, 0)),
            pl.BlockSpec((1, HH), lambda: (0, 0)),
            pl.BlockSpec((1, 1), lambda: (0, 0)),
        ],
        out_specs=pl.BlockSpec((1, GG), lambda: (0, 0)),
        out_shape=jax.ShapeDtypeStruct((1, GG), jnp.float32),
    )(gsum_p, batch2d, b2r, wlr, blr)


def kernel(x, edge_index, batch, W1, b1, W2, b2, Wl, bl):
    src = edge_index[0]
    dst = edge_index[1]
    x_p = jnp.pad(x, ((0, NP - NN), (0, 0)))
    batch_p = jnp.pad(batch, (0, NP - NN), constant_values=GG)
    z640 = jnp.zeros((PERT,), jnp.float32)
    zrow = jnp.zeros((CH, HH), jnp.float32)
    b1r = b1.reshape(1, HH)
    wlr = Wl.reshape(1, HH)
    b2r = b2.reshape(1, HH)
    blr = bl.reshape(1, 1)

    deg_p = _k1_deg(dst, z640)
    hd, dis2d = _k2(deg_p, x_p, W1)
    acc_p = _k3_msg(hd, src, dst, zrow)
    td2d = _k4(acc_p, hd, dis2d, b1r, W2, wlr)
    gsum_p = _k5_pool(td2d.reshape(NP), dis2d.reshape(NP), batch_p, src, dst, z640)
    out = _k6(gsum_p, batch_p.reshape(NP // 128, 128), b2r, wlr, blr)
    return out.reshape(GG, 1)


# speedup vs baseline: 24.6765x; 24.6765x over previous
import functools

import jax
import jax.numpy as jnp
from jax import lax
from jax.experimental import pallas as pl
from jax.experimental.pallas import tpu as pltpu
from jax.experimental.pallas import tpu_sc as plsc

NN = 10000
EE = 320000
DD = 128
HH = 32
GG = 64
NP = 10240
CH = 128
NBLK = EE // CH
NWORK = 32
PERT = NP // 16

_mesh = plsc.VectorSubcoreMesh(core_axis_name="c", subcore_axis_name="s")


def _wid(c, s):
    return s * 2 + c


def _nblk(wid):
    base = NBLK // NWORK
    rem = NBLK % NWORK
    return base + (wid < rem).astype(jnp.int32)


@functools.partial(
    pl.kernel,
    out_type=jax.ShapeDtypeStruct((2, NP), jnp.float32),
    mesh=_mesh,
    scratch_types=[
        pltpu.VMEM((CH,), jnp.int32),
        pltpu.VMEM((CH,), jnp.float32),
        pltpu.VMEM((PERT,), jnp.float32),
        pltpu.VMEM_SHARED((NP,), jnp.float32),
    ],
)
def _k1_deg(dst_hbm, z640_hbm, deg_out, idx_d, ones_v, zb, deg_s):
    c = lax.axis_index("c")
    s = lax.axis_index("s")
    wid = _wid(c, s)
    for i in range(CH // 16):
        ones_v[pl.ds(16 * i, 16)] = jnp.full((16,), 1.0, jnp.float32)
    pltpu.sync_copy(z640_hbm, zb)
    pltpu.sync_copy(zb, deg_s.at[pl.ds(s * PERT, PERT)])
    plsc.subcore_barrier()

    def body(j, carry):
        off = (wid + NWORK * j) * CH
        pltpu.sync_copy(dst_hbm.at[pl.ds(off, CH)], idx_d)
        pltpu.sync_copy(ones_v, deg_s.at[idx_d], add=True)
        return carry

    lax.fori_loop(0, _nblk(wid), body, 0)
    plsc.subcore_barrier()
    pltpu.sync_copy(deg_s.at[pl.ds(s * PERT, PERT)],
                    deg_out.at[c].at[pl.ds(s * PERT, PERT)])


def _k2a_body(d0_ref, d1_ref, disrepT_ref):
    deg = d0_ref[0] + d1_ref[0] + 1.0
    dis = lax.rsqrt(deg)
    disrepT_ref[...] = jnp.broadcast_to(dis, (HH, dis.shape[1]))


def _k2a(deg_p):
    rb = 1024
    grid = NP // rb
    deg3 = deg_p.reshape(2 * NP // rb, 1, rb)
    return pl.pallas_call(
        _k2a_body,
        grid=(grid,),
        in_specs=[
            pl.BlockSpec((1, 1, rb), lambda i: (i, 0, 0)),
            pl.BlockSpec((1, 1, rb), lambda i: (i + NP // rb, 0, 0)),
        ],
        out_specs=pl.BlockSpec((HH, rb), lambda i: (0, i)),
        out_shape=jax.ShapeDtypeStruct((HH, NP), jnp.float32),
    )(deg3, deg3)


def _k2b_body(x_ref, w1_ref, disrep_ref, hd_ref):
    h = jnp.dot(x_ref[...], w1_ref[...], preferred_element_type=jnp.float32)
    hd_ref[...] = h * disrep_ref[...]


def _k2b(x_p, w1, disrep):
    rb = 1024
    grid = NP // rb
    return pl.pallas_call(
        _k2b_body,
        grid=(grid,),
        in_specs=[
            pl.BlockSpec((rb, DD), lambda i: (i, 0)),
            pl.BlockSpec((DD, HH), lambda i: (0, 0)),
            pl.BlockSpec((rb, HH), lambda i: (i, 0)),
        ],
        out_specs=pl.BlockSpec((rb, HH), lambda i: (i, 0)),
        out_shape=jax.ShapeDtypeStruct((NP, HH), jnp.float32),
    )(x_p, w1, disrep)


@functools.partial(
    pl.kernel,
    out_type=jax.ShapeDtypeStruct((2, NP, HH), jnp.float32),
    mesh=_mesh,
    scratch_types=[
        pltpu.VMEM((CH,), jnp.int32),
        pltpu.VMEM((CH,), jnp.int32),
        pltpu.VMEM((CH, HH), jnp.float32),
        pltpu.VMEM((CH, HH), jnp.float32),
        pltpu.SemaphoreType.DMA,
        pltpu.VMEM_SHARED((NP, HH), jnp.float32),
    ],
    compiler_params=pltpu.CompilerParams(use_tc_tiling_on_sc=False),
)
def _k3_msg(hd_hbm, src_hbm, dst_hbm, zrow_hbm, acc_out,
            idx_s, idx_d, rows, zb, sem, acc_s):
    c = lax.axis_index("c")
    s = lax.axis_index("s")
    wid = _wid(c, s)
    pltpu.sync_copy(zrow_hbm, zb)
    for j in range(PERT // CH):
        pltpu.sync_copy(zb, acc_s.at[pl.ds(s * PERT + j * CH, CH)])
    plsc.subcore_barrier()

    def body(j, carry):
        off = (wid + NWORK * j) * CH
        pltpu.sync_copy(src_hbm.at[pl.ds(off, CH)], idx_s)
        pltpu.sync_copy(dst_hbm.at[pl.ds(off, CH)], idx_d)
        pltpu.async_copy(hd_hbm.at[idx_s], rows, sem).wait()
        pltpu.sync_copy(rows, acc_s.at[idx_d], add=True)
        return carry

    lax.fori_loop(0, _nblk(wid), body, 0)
    plsc.subcore_barrier()
    pltpu.sync_copy(acc_s.at[pl.ds(s * PERT, PERT)],
                    acc_out.at[c].at[pl.ds(s * PERT, PERT)])


def _k4_body(acc_ref, hd_ref, disrep_ref, b1_ref, w2_ref, wlr_ref, td_ref):
    disrep = disrep_ref[...]
    pre = (acc_ref[0] + acc_ref[1] + hd_ref[...]) * disrep + b1_ref[...]
    a1 = jnp.maximum(pre, 0.0)
    z = jnp.dot(a1, w2_ref[...], preferred_element_type=jnp.float32)
    t = jnp.sum(z * wlr_ref[...], axis=1, keepdims=True)
    td_ref[...] = t * disrep


def _k4(acc_p, hd, disrep, b1r, w2, wlr):
    rb = 1024
    grid = NP // rb
    return pl.pallas_call(
        _k4_body,
        grid=(grid,),
        in_specs=[
            pl.BlockSpec((2, rb, HH), lambda i: (0, i, 0)),
            pl.BlockSpec((rb, HH), lambda i: (i, 0)),
            pl.BlockSpec((rb, HH), lambda i: (i, 0)),
            pl.BlockSpec((1, HH), lambda i: (0, 0)),
            pl.BlockSpec((HH, HH), lambda i: (0, 0)),
            pl.BlockSpec((1, HH), lambda i: (0, 0)),
        ],
        out_specs=pl.BlockSpec((rb, HH), lambda i: (i, 0)),
        out_shape=jax.ShapeDtypeStruct((NP, HH), jnp.float32),
    )(acc_p, hd, disrep, b1r, w2, wlr)


@functools.partial(
    pl.kernel,
    out_type=jax.ShapeDtypeStruct((2, 2, 2 * GG), jnp.float32),
    mesh=_mesh,
    scratch_types=[
        pltpu.VMEM((CH,), jnp.int32),
        pltpu.VMEM((CH,), jnp.int32),
        pltpu.VMEM((CH,), jnp.float32),
        pltpu.VMEM((CH,), jnp.float32),
        pltpu.VMEM((CH,), jnp.float32),
        pltpu.VMEM((CH,), jnp.float32),
        pltpu.VMEM((CH,), jnp.float32),
        pltpu.VMEM((CH,), jnp.int32),
        pltpu.VMEM((NP,), jnp.float32),
        pltpu.VMEM((PERT,), jnp.float32),
        pltpu.VMEM_SHARED((NP,), jnp.float32),
        pltpu.VMEM_SHARED((2 * GG,), jnp.float32),
        pltpu.VMEM_SHARED((2 * GG,), jnp.float32),
    ],
    compiler_params=pltpu.CompilerParams(
        use_tc_tiling_on_sc=False, needs_layout_passes=False),
)
def _k5_pool(td_hbm, dis_hbm, batch_hbm, src_hbm, dst_hbm, z640_hbm, gsum_out,
             idx_s, idx_d, vals, qbuf, cbuf, rbuf, dbuf, bbuf, td_v, zb,
             r_s, bins_s, cbin_s):
    c = lax.axis_index("c")
    s = lax.axis_index("s")
    wid = _wid(c, s)
    pltpu.sync_copy(td_hbm, td_v)
    pltpu.sync_copy(z640_hbm, zb)
    pltpu.sync_copy(zb, r_s.at[pl.ds(s * PERT, PERT)])

    @pl.when(s == 0)
    def _():
        pltpu.sync_copy(zb.at[pl.ds(0, 2 * GG)], bins_s)

    @pl.when(s == 1)
    def _():
        pltpu.sync_copy(zb.at[pl.ds(0, 2 * GG)], cbin_s)

    plsc.subcore_barrier()

    def body(j, carry):
        off = (wid + NWORK * j) * CH
        pltpu.sync_copy(src_hbm.at[pl.ds(off, CH)], idx_s)
        pltpu.sync_copy(dst_hbm.at[pl.ds(off, CH)], idx_d)
        for f in range(CH // 16):
            sl = pl.ds(16 * f, 16)
            vals[sl] = plsc.load_gather(td_v, [idx_s[sl]])
        pltpu.sync_copy(vals, r_s.at[idx_d], add=True)
        return carry

    lax.fori_loop(0, _nblk(wid), body, 0)
    plsc.subcore_barrier()

    coef = jnp.where(c == 0, 1.0, 0.0).astype(jnp.float32)
    for f in range(CH // 16):
        cbuf[pl.ds(16 * f, 16)] = jnp.full((16,), 1.0, jnp.float32) * coef
    for k in range(PERT // CH):
        off = s * PERT + k * CH
        pltpu.sync_copy(r_s.at[pl.ds(off, CH)], rbuf)
        pltpu.sync_copy(dis_hbm.at[pl.ds(off, CH)], dbuf)
        pltpu.sync_copy(batch_hbm.at[pl.ds(off, CH)], bbuf)
        for f in range(CH // 16):
            sl = pl.ds(16 * f, 16)
            qbuf[sl] = dbuf[sl] * (rbuf[sl] + coef * td_v[pl.ds(off + 16 * f, 16)])
        pltpu.sync_copy(qbuf, bins_s.at[bbuf], add=True)
        pltpu.sync_copy(cbuf, cbin_s.at[bbuf], add=True)
    plsc.subcore_barrier()

    @pl.when(s == 0)
    def _():
        pltpu.sync_copy(bins_s, gsum_out.at[c].at[0])

    @pl.when(s == 1)
    def _():
        pltpu.sync_copy(cbin_s, gsum_out.at[c].at[1])


def _k6_body(gsum_ref, b2_ref, wlr_ref, bl_ref, out_ref):
    gs = gsum_ref[...]
    tot = gs[0, 0:1, :GG] + gs[1, 0:1, :GG]
    cnt = gs[0, 1:2, :GG] + gs[1, 1:2, :GG]
    sbw = jnp.sum(b2_ref[...] * wlr_ref[...])
    out = tot / jnp.maximum(cnt, 1.0) + jnp.where(cnt > 0.0, sbw, 0.0)
    out_ref[...] = out + bl_ref[...]


def _k6(gsum_p, b2r, wlr, blr):
    return pl.pallas_call(
        _k6_body,
        in_specs=[
            pl.BlockSpec((2, 2, 2 * GG), lambda: (0, 0, 0)),
            pl.BlockSpec((1, HH), lambda: (0, 0)),
            pl.BlockSpec((1, HH), lambda: (0, 0)),
            pl.BlockSpec((1, 1), lambda: (0, 0)),
        ],
        out_specs=pl.BlockSpec((1, GG), lambda: (0, 0)),
        out_shape=jax.ShapeDtypeStruct((1, GG), jnp.float32),
    )(gsum_p, b2r, wlr, blr)


def kernel(x, edge_index, batch, W1, b1, W2, b2, Wl, bl):
    src = edge_index[0]
    dst = edge_index[1]
    x_p = jnp.pad(x, ((0, NP - NN), (0, 0)))
    batch_p = jnp.pad(batch, (0, NP - NN), constant_values=GG)
    z640 = jnp.zeros((PERT,), jnp.float32)
    zrow = jnp.zeros((CH, HH), jnp.float32)
    b1r = b1.reshape(1, HH)
    wlr = Wl.reshape(1, HH)
    b2r = b2.reshape(1, HH)
    blr = bl.reshape(1, 1)

    deg_p = _k1_deg(dst, z640)
    disrep = _k2a(deg_p).T
    hd = _k2b(x_p, W1, disrep)
    acc_p = _k3_msg(hd, src, dst, zrow)
    tdrep = _k4(acc_p, hd, disrep, b1r, W2, wlr)
    gsum_p = _k5_pool(tdrep[:, 0], disrep[:, 0], batch_p, src, dst, z640)
    out = _k6(gsum_p, b2r, wlr, blr)
    return out.reshape(GG, 1)
